# Initial kernel scaffold; baseline (speedup 1.0000x reference)
import jax
import jax.numpy as jnp
from jax.experimental import pallas as pl


def kernel(data, weights):
    raise NotImplementedError("write your pallas kernel here")


# speedup vs baseline: 33.7794x; 33.7794x over previous
import functools

import jax
import jax.numpy as jnp
from jax import lax
from jax.experimental import pallas as pl
from jax.experimental.pallas import tpu as pltpu
from jax.experimental.pallas import tpu_sc as plsc

NUM_CATEGORIES = 100000
N = 8388608

_NC = 2
_NS = 16
_NW = _NC * _NS
_PER = N // _NW
_CHUNK = 4096
_NCHUNK = _PER // _CHUNK
_L = 16
_HPAD = 102400


def _sc_hist_body(data_hbm, weights_hbm, out_hbm, hist, dbuf, wbuf):
    wid = lax.axis_index("s") * _NC + lax.axis_index("c")
    base = wid * _PER

    zeros = jnp.zeros((_L,), jnp.float32)

    def zero_body(i, carry):
        hist[pl.ds(i * _L, _L)] = zeros
        return carry

    lax.fori_loop(0, _HPAD // _L, zero_body, 0)

    def chunk_body(g, carry):
        off = base + g * _CHUNK
        pltpu.sync_copy(data_hbm.at[pl.ds(off, _CHUNK)], dbuf)
        pltpu.sync_copy(weights_hbm.at[pl.ds(off, _CHUNK)], wbuf)

        def inner(i, c2):
            idx = dbuf[pl.ds(i * _L, _L)]
            w = wbuf[pl.ds(i * _L, _L)]
            plsc.addupdate_scatter(hist, [idx], w)
            return c2

        lax.fori_loop(0, _CHUNK // _L, inner, 0)
        return carry

    lax.fori_loop(0, _NCHUNK, chunk_body, 0)

    pltpu.sync_copy(hist, out_hbm.at[wid])


_sc_hist = functools.partial(
    pl.kernel,
    out_type=jax.ShapeDtypeStruct((_NW, _HPAD), jnp.float32),
    mesh=plsc.VectorSubcoreMesh(core_axis_name="c", subcore_axis_name="s"),
    compiler_params=pltpu.CompilerParams(needs_layout_passes=False),
    scratch_types=[
        pltpu.VMEM((_HPAD,), jnp.float32),
        pltpu.VMEM((_CHUNK,), jnp.int32),
        pltpu.VMEM((_CHUNK,), jnp.float32),
    ],
)(_sc_hist_body)


def _tc_reduce_body(parts_ref, out_ref):
    s = jnp.sum(parts_ref[...], axis=0, keepdims=True)
    total = jnp.sum(s)
    out_ref[...] = jnp.log(s / total)


_tc_reduce = pl.pallas_call(
    _tc_reduce_body,
    out_shape=jax.ShapeDtypeStruct((1, _HPAD), jnp.float32),
)


def kernel(data, weights):
    parts = _sc_hist(data, weights)
    logp = _tc_reduce(parts)
    return logp[0, :NUM_CATEGORIES]


# speedup vs baseline: 56.9988x; 1.6874x over previous
import functools

import jax
import jax.numpy as jnp
from jax import lax
from jax.experimental import pallas as pl
from jax.experimental.pallas import tpu as pltpu
from jax.experimental.pallas import tpu_sc as plsc

NUM_CATEGORIES = 100000
N = 8388608

_NC = 2
_NS = 16
_NW = _NC * _NS
_PER = N // _NW
_CHUNK = 4096
_NCHUNK = _PER // _CHUNK
_L = 16
_HPAD = 102400
_UNROLL = 16


def _sc_hist_body(data_hbm, weights_hbm, out_hbm, hist, dbuf, wbuf,
                  sd0, sw0, sd1, sw1):
    wid = lax.axis_index("s") * _NC + lax.axis_index("c")
    base = wid * _PER
    sems = (sd0, sw0, sd1, sw1)

    zeros = jnp.zeros((_L,), jnp.float32)

    def zero_body(i, carry):
        for j in range(8):
            hist[pl.ds(i * (8 * _L) + j * _L, _L)] = zeros
        return carry

    lax.fori_loop(0, _HPAD // (8 * _L), zero_body, 0)

    def start_fetch(slot, c):
        off = jnp.minimum(base + c * _CHUNK, N - _CHUNK)
        pltpu.async_copy(data_hbm.at[pl.ds(off, _CHUNK)],
                         dbuf.at[slot], sems[2 * slot])
        pltpu.async_copy(weights_hbm.at[pl.ds(off, _CHUNK)],
                         wbuf.at[slot], sems[2 * slot + 1])

    def wait_fetch(slot):
        pltpu.make_async_copy(data_hbm.at[pl.ds(0, _CHUNK)], dbuf.at[slot],
                              sems[2 * slot]).wait()
        pltpu.make_async_copy(weights_hbm.at[pl.ds(0, _CHUNK)], wbuf.at[slot],
                              sems[2 * slot + 1]).wait()

    def scatter_chunk(slot):
        def inner(i, c2):
            for j in range(_UNROLL):
                s = pl.ds(i * (_UNROLL * _L) + j * _L, _L)
                plsc.addupdate_scatter(hist, [dbuf[slot, s]], wbuf[slot, s])
            return c2

        lax.fori_loop(0, _CHUNK // (_UNROLL * _L), inner, 0)

    start_fetch(0, 0)
    start_fetch(1, 1)

    def pair_body(t, carry):
        c = 2 * t
        wait_fetch(0)
        scatter_chunk(0)
        start_fetch(0, c + 2)
        wait_fetch(1)
        scatter_chunk(1)
        start_fetch(1, c + 3)
        return carry

    lax.fori_loop(0, _NCHUNK // 2 - 1, pair_body, 0)

    wait_fetch(0)
    scatter_chunk(0)
    wait_fetch(1)
    scatter_chunk(1)

    pltpu.sync_copy(hist, out_hbm.at[wid])


_sc_hist = functools.partial(
    pl.kernel,
    out_type=jax.ShapeDtypeStruct((_NW, _HPAD), jnp.float32),
    mesh=plsc.VectorSubcoreMesh(core_axis_name="c", subcore_axis_name="s"),
    compiler_params=pltpu.CompilerParams(needs_layout_passes=False),
    scratch_types=[
        pltpu.VMEM((_HPAD,), jnp.float32),
        pltpu.VMEM((2, _CHUNK), jnp.int32),
        pltpu.VMEM((2, _CHUNK), jnp.float32),
        pltpu.SemaphoreType.DMA,
        pltpu.SemaphoreType.DMA,
        pltpu.SemaphoreType.DMA,
        pltpu.SemaphoreType.DMA,
    ],
)(_sc_hist_body)


def _tc_reduce_body(parts_ref, out_ref):
    s = jnp.sum(parts_ref[...], axis=0, keepdims=True)
    total = jnp.sum(s)
    out_ref[...] = jnp.log(s / total)


_tc_reduce = pl.pallas_call(
    _tc_reduce_body,
    out_shape=jax.ShapeDtypeStruct((1, _HPAD), jnp.float32),
)


def kernel(data, weights):
    parts = _sc_hist(data, weights)
    logp = _tc_reduce(parts)
    return logp[0, :NUM_CATEGORIES]


# speedup vs baseline: 89.0233x; 1.5618x over previous
import functools

import jax
import jax.numpy as jnp
from jax import lax
from jax.experimental import pallas as pl
from jax.experimental.pallas import tpu as pltpu
from jax.experimental.pallas import tpu_sc as plsc

NUM_CATEGORIES = 100000
N = 8388608

_NC = 2
_NS = 16
_NW = _NC * _NS
_PER = N // _NW
_CHUNK = 4096
_NCHUNK = _PER // _CHUNK
_L = 16
_HPAD = 102400
_UNROLL = 16


def _sc_hist_body(data_hbm, weights_hbm, out_hbm, hist, dbuf, wbuf,
                  sd0, sw0, sd1, sw1):
    wid = lax.axis_index("s") * _NC + lax.axis_index("c")
    base = wid * _PER
    sems = (sd0, sw0, sd1, sw1)

    zeros = jnp.zeros((_L,), jnp.float32)

    @plsc.parallel_loop(0, _HPAD, step=_L, unroll=8)
    def _zero_body(i):
        hist[pl.ds(i, _L)] = zeros

    def start_fetch(slot, c):
        off = jnp.minimum(base + c * _CHUNK, N - _CHUNK)
        pltpu.async_copy(data_hbm.at[pl.ds(off, _CHUNK)],
                         dbuf.at[slot], sems[2 * slot])
        pltpu.async_copy(weights_hbm.at[pl.ds(off, _CHUNK)],
                         wbuf.at[slot], sems[2 * slot + 1])

    def wait_fetch(slot):
        pltpu.make_async_copy(data_hbm.at[pl.ds(0, _CHUNK)], dbuf.at[slot],
                              sems[2 * slot]).wait()
        pltpu.make_async_copy(weights_hbm.at[pl.ds(0, _CHUNK)], wbuf.at[slot],
                              sems[2 * slot + 1]).wait()

    def scatter_chunk(slot):
        @plsc.parallel_loop(0, _CHUNK, step=_L, unroll=_UNROLL)
        def _scatter_body(i):
            s = pl.ds(i, _L)
            plsc.addupdate_scatter(hist, [dbuf[slot, s]], wbuf[slot, s])

    start_fetch(0, 0)
    start_fetch(1, 1)

    def pair_body(t, carry):
        c = 2 * t
        wait_fetch(0)
        scatter_chunk(0)
        start_fetch(0, c + 2)
        wait_fetch(1)
        scatter_chunk(1)
        start_fetch(1, c + 3)
        return carry

    lax.fori_loop(0, _NCHUNK // 2 - 1, pair_body, 0)

    wait_fetch(0)
    scatter_chunk(0)
    wait_fetch(1)
    scatter_chunk(1)

    pltpu.sync_copy(hist, out_hbm.at[wid])


_sc_hist = functools.partial(
    pl.kernel,
    out_type=jax.ShapeDtypeStruct((_NW, _HPAD), jnp.float32),
    mesh=plsc.VectorSubcoreMesh(core_axis_name="c", subcore_axis_name="s"),
    compiler_params=pltpu.CompilerParams(needs_layout_passes=False),
    scratch_types=[
        pltpu.VMEM((_HPAD,), jnp.float32),
        pltpu.VMEM((2, _CHUNK), jnp.int32),
        pltpu.VMEM((2, _CHUNK), jnp.float32),
        pltpu.SemaphoreType.DMA,
        pltpu.SemaphoreType.DMA,
        pltpu.SemaphoreType.DMA,
        pltpu.SemaphoreType.DMA,
    ],
)(_sc_hist_body)


def _tc_reduce_body(parts_ref, out_ref):
    s = jnp.sum(parts_ref[...], axis=0, keepdims=True)
    total = jnp.sum(s)
    out_ref[...] = jnp.log(s / total)


_tc_reduce = pl.pallas_call(
    _tc_reduce_body,
    out_shape=jax.ShapeDtypeStruct((1, _HPAD), jnp.float32),
)


def kernel(data, weights):
    parts = _sc_hist(data, weights)
    logp = _tc_reduce(parts)
    return logp[0, :NUM_CATEGORIES]
